# baseline (speedup 1.0000x reference)
import functools

import jax
import jax.numpy as jnp
import numpy as np
from jax.experimental import pallas as pl
from jax.experimental.pallas import tpu as pltpu

N = 10000
NMOL = 512
HIDDEN = 64
FILTERS = 64
NG = 50
NGP = 64
T = 6
CUTOFF = 10.0

R = 64
C = 128
CH = 64
NP = 10112
NPA = NP + C
NT = NP // R
NTA = NPA // R

_LOG2 = np.float32(np.log(2.0))


def _sp(v):
    return jnp.log1p(jnp.exp(v))


def _body(coeff_ref, rowpack_ref, batchrow_ref, offs_ref,
          embp_ref, arefp_ref,
          w1_ref, b1_ref, w2_ref, b2_ref, lin1_ref, lin2_ref, lin2b_ref,
          linw_ref, linb_ref, o1_ref, o1b_ref, o2_ref, o2b_ref,
          out_ref, h2_ref, bounds_ref):
    coeff = coeff_ref[0]
    offs = offs_ref[:, :, :]
    half_hi = jax.lax.broadcasted_iota(jnp.int32, (1, 1, 128), 2) >= CH
    iota_l = jax.lax.broadcasted_iota(jnp.int32, (1, C), 1).astype(jnp.float32)
    iota_s = jax.lax.broadcasted_iota(jnp.int32, (R, 1), 0).astype(jnp.float32)
    iota_cls = jax.lax.broadcasted_iota(jnp.int32, (1, 128), 1).astype(jnp.float32)
    iota_mol = jax.lax.broadcasted_iota(jnp.int32, (1, NMOL), 1).astype(jnp.float32)
    batch_row = batchrow_ref[:, :]

    def init_tile(r, _):
        sl = pl.ds(r * R, R)
        xr = rowpack_ref[sl, 4:5]
        onehot = (xr == iota_cls).astype(jnp.float32)
        h2_ref[0, sl, :] = jnp.dot(onehot, embp_ref[:, :],
                                   preferred_element_type=jnp.float32)
        h2_ref[1, sl, :] = jnp.zeros((R, HIDDEN), jnp.float32)
        return 0

    jax.lax.fori_loop(0, NTA, init_tile, 0)

    def bounds_tile(r, _):
        br = rowpack_ref[pl.ds(r * R, R), 3:4]
        b_lo = jnp.min(br)
        b_hi = jnp.max(br)
        cnt_lo = jnp.sum((batch_row < b_lo).astype(jnp.int32))
        cnt_hi = jnp.sum((batch_row <= b_hi).astype(jnp.int32))
        cw = (cnt_lo // 8) * 8
        bounds_ref[r, 0] = cw
        bounds_ref[r, 1] = (cnt_hi - cw + C - 1) // C
        return 0

    jax.lax.fori_loop(0, NT, bounds_tile, 0)

    def layer(t, _):
        p = jax.lax.rem(t, 2)
        wl1 = lin1_ref[t]
        w1 = w1_ref[t]
        b1 = b1_ref[t]
        w2 = w2_ref[t]
        b2 = b2_ref[t]
        wl2 = lin2_ref[t]
        bl2 = lin2b_ref[t]
        wl = linw_ref[t]
        bl = linb_ref[t]

        def row_tile(r, _):
            sl = pl.ds(r * R, R)
            rp = rowpack_ref[sl, :]
            px = rp[:, 0:1]
            py = rp[:, 1:2]
            pz = rp[:, 2:3]
            br = rp[:, 3:4]
            cw = bounds_ref[r, 0]
            nw = bounds_ref[r, 1]
            gi = iota_s + (r * R).astype(jnp.float32)

            def col_step(k, acc):
                c0 = cw + k * C
                cp = jnp.transpose(rowpack_ref[pl.ds(c0, C), :])
                dx = px - cp[0:1, :]
                dy = py - cp[1:2, :]
                dz = pz - cp[2:3, :]
                d2 = dx * dx + dy * dy + dz * dz
                d = jnp.sqrt(d2 + 1e-12)
                gj = iota_l + c0.astype(jnp.float32)
                mask = ((d2 <= CUTOFF * CUTOFF)
                        & (br == cp[3:4, :])
                        & (gi != gj))
                env = 0.5 * (jnp.cos(d * jnp.pi / CUTOFF) + 1.0)
                scale = jnp.where(mask, env, 0.0)
                d3 = d[:, :, None]
                d_sel = jnp.where(half_hi, d3[:, CH:, :], d3[:, :CH, :])
                dd = d_sel - offs
                rbf = jnp.exp2(coeff * dd * dd)
                rbf2 = rbf.reshape(R * CH, 128)
                s = jnp.log2(1.0 + jnp.exp2(
                    jnp.dot(rbf2, w1, preferred_element_type=jnp.float32)
                    + b1))
                W = jnp.dot(s, w2, preferred_element_type=jnp.float32) + b2
                W3 = W.reshape(R, CH, 128)
                sc3 = scale[:, :, None]
                sc_sel = jnp.where(half_hi, sc3[:, CH:, :], sc3[:, :CH, :])
                hxc = jnp.dot(h2_ref[p, pl.ds(c0, C), :], wl1,
                              preferred_element_type=jnp.float32)
                hx2 = jnp.concatenate([hxc[:CH, :], hxc[CH:, :]], axis=1)
                term = W3 * sc_sel * hx2[None, :, :]
                return acc + jnp.sum(term, axis=1)

            acc2 = jax.lax.fori_loop(0, nw, col_step,
                                     jnp.zeros((R, 128), jnp.float32))
            acc = acc2[:, :FILTERS] + acc2[:, FILTERS:]
            v = _sp(jnp.dot(acc, wl2, preferred_element_type=jnp.float32)
                     + bl2)
            v = jnp.dot(v, wl, preferred_element_type=jnp.float32) + bl
            h2_ref[1 - p, sl, :] = h2_ref[p, sl, :] + v
            return 0

        jax.lax.fori_loop(0, NT, row_tile, 0, unroll=2)
        return 0

    jax.lax.fori_loop(0, T, layer, 0)

    def readout(r, eacc):
        sl = pl.ds(r * R, R)
        ht = h2_ref[T % 2, sl, :]
        hh = _sp(jnp.dot(ht, o1_ref[:, :],
                          preferred_element_type=jnp.float32) + o1b_ref[:, :])
        e8 = jnp.dot(hh, o2_ref[:, :],
                     preferred_element_type=jnp.float32) + o2b_ref[:, :]
        xr = rowpack_ref[sl, 4:5]
        onehot = (xr == iota_cls).astype(jnp.float32)
        aref = jnp.dot(onehot, arefp_ref[:, :],
                       preferred_element_type=jnp.float32)
        e = e8[:, 0:1] + aref[:, 0:1]
        br = rowpack_ref[sl, 3:4]
        ohb = (br == iota_mol).astype(jnp.float32)
        return eacc + jnp.sum(ohb * e, axis=0, keepdims=True)

    eacc = jax.lax.fori_loop(0, NT, readout,
                             jnp.zeros((1, NMOL), jnp.float32))
    out_ref[:, :] = eacc


@functools.partial(jax.jit, static_argnums=())
def kernel(x, pos, batch, emb, atomref, mlp_w1, mlp_b1, mlp_w2, mlp_b2,
           lin1_w, lin2_w, lin2b, lin_w, lin_b, o1_w, o1_b, o2_w, o2_b):
    n = pos.shape[0]
    posf = pos.astype(jnp.float32)
    batchf = batch.astype(jnp.float32)
    xf = x.astype(jnp.float32)
    rowpack = jnp.zeros((NPA, 8), jnp.float32)
    rowpack = rowpack.at[:n, 0:3].set(posf)
    rowpack = rowpack.at[:n, 3].set(batchf)
    rowpack = rowpack.at[:n, 4].set(xf)
    rowpack = rowpack.at[n:, 3].set(float(NMOL))
    rowpack = rowpack.at[n:, 4].set(127.0)
    batch_row = rowpack[:NP, 3].reshape(1, NP)

    offsets = jnp.linspace(0.0, CUTOFF, NG)
    coeff = -0.5 / (offsets[1] - offsets[0]) ** 2
    offs1 = jnp.full((NGP,), 1e4, jnp.float32).at[:NG].set(offsets)
    offs3 = jnp.concatenate([offs1, offs1]).reshape(1, 1, 128)

    embp = jnp.zeros((128, HIDDEN), jnp.float32).at[:100].set(emb)
    arefp = jnp.zeros((128, 8), jnp.float32).at[:100, 0].set(atomref[:, 0])
    log2e = jnp.float32(1.0) / _LOG2
    w1s = mlp_w1 * log2e
    w2s = mlp_w2 * _LOG2
    w1p = jnp.zeros((T, 128, 128), jnp.float32)
    w1p = w1p.at[:, :NG, :FILTERS].set(w1s)
    w1p = w1p.at[:, NGP:NGP + NG, FILTERS:].set(w1s)
    b1s = mlp_b1 * log2e
    b1p = jnp.concatenate([b1s, b1s], axis=1)[:, None, :]
    w2p = jnp.zeros((T, 128, 128), jnp.float32)
    w2p = w2p.at[:, :FILTERS, :FILTERS].set(w2s)
    w2p = w2p.at[:, FILTERS:, FILTERS:].set(w2s)
    b2eff = mlp_b2 - _LOG2 * mlp_w2.sum(axis=1)
    b2p = jnp.concatenate([b2eff, b2eff], axis=1)[:, None, :]
    lin2bp = lin2b[:, None, :]
    linbp = (lin_b - _LOG2 * lin_w.sum(axis=1))[:, None, :]
    o1bp = o1_b[None, :]
    o2p = jnp.zeros((HIDDEN // 2, 8), jnp.float32).at[:, 0].set(o2_w[:, 0])
    o2beff = o2_b[0] - _LOG2 * o2_w[:, 0].sum()
    o2bp = jnp.zeros((1, 8), jnp.float32).at[0, 0].set(o2beff)
    coeffarr = jnp.reshape((coeff * log2e).astype(jnp.float32), (1,))

    energy2d = pl.pallas_call(
        _body,
        out_shape=jax.ShapeDtypeStruct((1, NMOL), jnp.float32),
        in_specs=[
            pl.BlockSpec(memory_space=pltpu.SMEM),
        ] + [pl.BlockSpec(memory_space=pltpu.VMEM)] * 18,
        out_specs=pl.BlockSpec(memory_space=pltpu.VMEM),
        scratch_shapes=[
            pltpu.VMEM((2, NPA, HIDDEN), jnp.float32),
            pltpu.SMEM((NT, 2), jnp.int32),
        ],
    )(coeffarr, rowpack, batch_row, offs3, embp, arefp,
      w1p, b1p, w2p, b2p, lin1_w, lin2_w, lin2bp, lin_w, linbp,
      o1_w, o1bp, o2p, o2bp)
    return energy2d[0]
